# baseline (speedup 1.0000x reference)
import functools

import numpy as _np

import jax
import jax.numpy as jnp
from jax import lax
from jax.experimental import pallas as pl
from jax.experimental.pallas import tpu as pltpu
from jax.experimental.pallas import tpu_sc as plsc

N = 10000
D = 128
E = 320000
B = 64

NC = 2
NS = 16
NW = NC * NS
CH = 128
NCH = 80
EPT = NCH * CH
PADE = NW * EPT
NP = 10240
RPT = NP // NS
ZR = 128

_mesh = plsc.VectorSubcoreMesh(core_axis_name="c", subcore_axis_name="s")

_PAD_SRC = _np.arange(PADE - E, dtype=_np.int32) % N
_PAD_DST = (N + _np.arange(PADE - E, dtype=_np.int32) % (NP - N)).astype(_np.int32)


def _zero_fill(ref, rows, width):
    per_row = width // 16

    def body(k, _):
        i = k // per_row
        j = k % per_row
        ref[i, pl.ds(j * 16, 16)] = jnp.zeros((16,), jnp.float32)
        return 0

    lax.fori_loop(0, rows * per_row, body, 0)


def _deg_body(dst2, degp, dst_all, ones_v, zbuf, acc, sem_d, sem_c):
    cid = lax.axis_index("c")
    sid = lax.axis_index("s")
    wid = cid * NS + sid

    pltpu.async_copy(dst2.at[pl.ds(wid * NCH, NCH), :], dst_all, sem_d)
    _zero_fill(zbuf, ZR, D)

    def ones_body(k, _):
        ones_v[k // 8, pl.ds((k % 8) * 16, 16)] = jnp.ones((16,), jnp.float32)
        return 0

    lax.fori_loop(0, CH * 8, ones_body, 0)
    for z in range(RPT // ZR):
        pltpu.sync_copy(zbuf, acc.at[pl.ds(sid * RPT + z * ZR, ZR)])
    plsc.subcore_barrier()
    pltpu.make_async_copy(dst2.at[pl.ds(wid * NCH, NCH), :], dst_all,
                          sem_d).wait()

    def grp(g, _):
        for k in range(8):
            pltpu.async_copy(ones_v, acc.at[dst_all.at[g * 8 + k]], sem_c,
                             add=True)
        for k in range(8):
            pltpu.make_async_copy(ones_v, acc.at[dst_all.at[g * 8 + k]],
                                  sem_c).wait()
        return 0

    lax.fori_loop(0, NCH // 8, grp, 0)

    plsc.subcore_barrier()
    for z in range(RPT // ZR):
        r0 = sid * RPT + z * ZR
        pltpu.sync_copy(acc.at[pl.ds(r0, ZR)], degp.at[cid, pl.ds(r0, ZR)])


_deg_call = pl.kernel(
    _deg_body,
    out_type=jax.ShapeDtypeStruct((NC, NP, D), jnp.float32),
    mesh=_mesh,
    scratch_types=[
        pltpu.VMEM((NCH, CH), jnp.int32),
        pltpu.VMEM((CH, D), jnp.float32),
        pltpu.VMEM((ZR, D), jnp.float32),
        pltpu.VMEM_SHARED((NP, D), jnp.float32),
        pltpu.SemaphoreType.DMA,
        pltpu.SemaphoreType.DMA,
    ],
)


def _agg_body(gp, src1, dst2, p, src_v0, src_v1, dst_all, rows0, rows1, acc,
              sem_d, sem_i0, sem_i1, sem_g0, sem_g1, sem_c0, sem_c1):
    cid = lax.axis_index("c")
    sid = lax.axis_index("s")
    wid = cid * NS + sid
    cb = wid * NCH
    eb = wid * EPT

    src_v = (src_v0, src_v1)
    sem_i = (sem_i0, sem_i1)
    rows = (rows0, rows1)
    sem_g = (sem_g0, sem_g1)
    sem_c = (sem_c0, sem_c1)

    def i_start(k, p_):
        pltpu.async_copy(src1.at[pl.ds(eb + k * CH, CH)], src_v[p_],
                         sem_i[p_])

    def i_wait(k, p_):
        pltpu.make_async_copy(src1.at[pl.ds(eb + k * CH, CH)], src_v[p_],
                              sem_i[p_]).wait()

    def g_start(k, p_):
        pltpu.async_copy(gp.at[src_v[p_]], rows[p_], sem_g[p_])

    def g_wait(k, p_):
        pltpu.make_async_copy(gp.at[src_v[p_]], rows[p_], sem_g[p_]).wait()

    def c_start(k, p_):
        pltpu.async_copy(rows[p_], acc.at[dst_all.at[k]], sem_c[p_], add=True)

    def c_wait(k, p_):
        pltpu.make_async_copy(rows[p_], acc.at[dst_all.at[k]],
                              sem_c[p_]).wait()

    pltpu.async_copy(dst2.at[pl.ds(cb, NCH), :], dst_all, sem_d)
    i_start(0, 0)
    _zero_fill(rows0, CH, D)
    for z in range(RPT // ZR):
        pltpu.sync_copy(rows0, acc.at[pl.ds(sid * RPT + z * ZR, ZR)])
    plsc.subcore_barrier()
    pltpu.make_async_copy(dst2.at[pl.ds(cb, NCH), :], dst_all, sem_d).wait()

    def c_sync(k, p_):
        pltpu.sync_copy(rows[p_], acc.at[dst_all.at[k]], add=True)

    i_wait(0, 0)
    g_start(0, 0)
    i_start(1, 1)
    g_wait(0, 0)
    i_wait(1, 1)
    g_start(1, 1)
    i_start(2, 0)
    c_sync(0, 0)

    def step(k, p_):
        g_wait(k, p_)
        i_wait(k + 1, 1 - p_)
        g_start(k + 1, 1 - p_)
        i_start(k + 2, p_)
        c_sync(k, p_)

    def body(g_, _):
        step(2 * g_ + 1, 1)
        step(2 * g_ + 2, 0)
        return 0

    lax.fori_loop(0, (NCH - 4) // 2, body, 0)
    step(NCH - 3, 1)
    k = NCH - 2
    g_wait(k, 0)
    i_wait(k + 1, 1)
    g_start(k + 1, 1)
    c_sync(k, 0)
    k = NCH - 1
    g_wait(k, 1)
    c_sync(k, 1)

    plsc.subcore_barrier()
    for z in range(RPT // ZR):
        r0 = sid * RPT + z * ZR
        pltpu.sync_copy(acc.at[pl.ds(r0, ZR)], p.at[cid, pl.ds(r0, ZR)])


_agg_call = pl.kernel(
    _agg_body,
    out_type=jax.ShapeDtypeStruct((NC, NP, D), jnp.float32),
    mesh=_mesh,
    scratch_types=[
        pltpu.VMEM((CH,), jnp.int32),
        pltpu.VMEM((CH,), jnp.int32),
        pltpu.VMEM((NCH, CH), jnp.int32),
        pltpu.VMEM((CH, D), jnp.float32),
        pltpu.VMEM((CH, D), jnp.float32),
        pltpu.VMEM_SHARED((NP, D), jnp.float32),
        pltpu.SemaphoreType.DMA,
        pltpu.SemaphoreType.DMA,
        pltpu.SemaphoreType.DMA,
        pltpu.SemaphoreType.DMA,
        pltpu.SemaphoreType.DMA,
        pltpu.SemaphoreType.DMA,
        pltpu.SemaphoreType.DMA,
    ],
)

MB = 1000


def _k0_body(x_ref, w_ref, degp_ref, gp_ref, dinv_ref):
    d = degp_ref[...]
    deg = 1.0 + d[0, :, 0:1] + d[1, :, 0:1]
    dinv = lax.rsqrt(deg)
    g = jnp.dot(x_ref[...], w_ref[...], preferred_element_type=jnp.float32)
    gp_ref[...] = g * dinv
    dinv_ref[...] = dinv


def _k0_call(x, w, degp):
    return pl.pallas_call(
        _k0_body,
        grid=(N // MB,),
        in_specs=[
            pl.BlockSpec((MB, D), lambda i: (i, 0)),
            pl.BlockSpec((D, D), lambda i: (0, 0)),
            pl.BlockSpec((NC, MB, D), lambda i: (0, i, 0)),
        ],
        out_specs=[
            pl.BlockSpec((MB, D), lambda i: (i, 0)),
            pl.BlockSpec((MB, 1), lambda i: (i, 0)),
        ],
        out_shape=[
            jax.ShapeDtypeStruct((N, D), jnp.float32),
            jax.ShapeDtypeStruct((N, 1), jnp.float32),
        ],
    )(x, w, degp)


def _mid_body(p_ref, gp_ref, dinv_ref, b_ref, w_ref, out_ref):
    pr = p_ref[...]
    dinv = dinv_ref[...]
    h = dinv * (pr[0] + pr[1] + gp_ref[...]) + b_ref[...]
    h = jnp.maximum(h, 0.0)
    out_ref[...] = dinv * jnp.dot(h, w_ref[...],
                                  preferred_element_type=jnp.float32)


def _mid_call(p, gp, dinv, b2d, w):
    return pl.pallas_call(
        _mid_body,
        grid=(N // MB,),
        in_specs=[
            pl.BlockSpec((NC, MB, D), lambda i: (0, i, 0)),
            pl.BlockSpec((MB, D), lambda i: (i, 0)),
            pl.BlockSpec((MB, 1), lambda i: (i, 0)),
            pl.BlockSpec((1, D), lambda i: (0, 0)),
            pl.BlockSpec((D, D), lambda i: (0, 0)),
        ],
        out_specs=pl.BlockSpec((MB, D), lambda i: (i, 0)),
        out_shape=jax.ShapeDtypeStruct((N, D), jnp.float32),
    )(p, gp, dinv, b2d, w)


PB = 200
PG = N // PB


def _pool_body(p_ref, gp_ref, dinv_ref, b_ref, batch_ref, wout_ref, bout_ref,
               out_ref, gmax_s, gsum_s, cnt_s):
    i = pl.program_id(0)

    @pl.when(i == 0)
    def _init():
        gmax_s[...] = jnp.full((B, D), -jnp.inf, jnp.float32)
        gsum_s[...] = jnp.zeros((B, D), jnp.float32)
        cnt_s[...] = jnp.zeros((B, 1), jnp.float32)

    pr = p_ref[...]
    dinv = dinv_ref[...]
    h = dinv * (pr[0] + pr[1] + gp_ref[...]) + b_ref[...]
    h = jnp.maximum(h, 0.0)

    bbc = batch_ref[0]
    gids = lax.broadcasted_iota(jnp.int32, (PB, B), 1)
    onehot = (gids == bbc).astype(jnp.float32)
    gsum_s[...] += lax.dot_general(
        onehot, h, (((0,), (0,)), ((), ())),
        preferred_element_type=jnp.float32)
    cnt_s[...] += jnp.sum(onehot, axis=0)[:, None]

    lo = bbc[0, 0]
    hi = bbc[PB - 1, 0]

    def seg(g, _):
        mask = bbc == g
        m = jnp.max(jnp.where(mask, h, -jnp.inf), axis=0, keepdims=True)
        cur = gmax_s[pl.ds(g, 1), :]
        gmax_s[pl.ds(g, 1), :] = jnp.maximum(cur, m)
        return 0

    lax.fori_loop(lo, hi + 1, seg, 0)

    @pl.when(i == PG - 1)
    def _final():
        gmean = gsum_s[...] / jnp.maximum(cnt_s[...], 1.0)
        pooled = jnp.concatenate([gmax_s[...], gmean], axis=1)
        out_ref[...] = jnp.dot(pooled, wout_ref[...],
                               preferred_element_type=jnp.float32) + bout_ref[...]


def _pool_call(p, gp, dinv, b2d, batch3d, wout, bout2d, out_dim):
    return pl.pallas_call(
        _pool_body,
        grid=(PG,),
        in_specs=[
            pl.BlockSpec((NC, PB, D), lambda i: (0, i, 0)),
            pl.BlockSpec((PB, D), lambda i: (i, 0)),
            pl.BlockSpec((PB, 1), lambda i: (i, 0)),
            pl.BlockSpec((1, D), lambda i: (0, 0)),
            pl.BlockSpec((1, PB, 1), lambda i: (i, 0, 0)),
            pl.BlockSpec((2 * D, out_dim), lambda i: (0, 0)),
            pl.BlockSpec((1, out_dim), lambda i: (0, 0)),
        ],
        out_specs=pl.BlockSpec((B, out_dim), lambda i: (0, 0)),
        out_shape=jax.ShapeDtypeStruct((B, out_dim), jnp.float32),
        scratch_shapes=[
            pltpu.VMEM((B, D), jnp.float32),
            pltpu.VMEM((B, D), jnp.float32),
            pltpu.VMEM((B, 1), jnp.float32),
        ],
    )(p, gp, dinv, b2d, batch3d, wout, bout2d)


def kernel(x, edge_index, batch_index, W0, b0, W1, b1, W2, b2, W3, b3,
           Wout, bout):
    out_dim = Wout.shape[1]
    batch3d = batch_index.reshape(PG, PB, 1)

    pad = PADE - E
    pad_dst = N + (jnp.arange(pad, dtype=jnp.int32) % (NP - N))
    dst2 = jnp.concatenate(
        [edge_index[1], pad_dst]).reshape(PADE // CH, CH)
    pad_src = jnp.arange(pad, dtype=jnp.int32) % N
    src1 = jnp.concatenate([edge_index[0], pad_src])
    degp = _deg_call(dst2)
    gp0, dinv = _k0_call(x, W0, degp)
    p0 = _agg_call(gp0, src1, dst2)
    gp1 = _mid_call(p0, gp0, dinv, b0.reshape(1, D), W1)
    p1 = _agg_call(gp1, src1, dst2)
    gp2 = _mid_call(p1, gp1, dinv, b1.reshape(1, D), W2)
    p2 = _agg_call(gp2, src1, dst2)
    gp3 = _mid_call(p2, gp2, dinv, b2.reshape(1, D), W3)
    p3 = _agg_call(gp3, src1, dst2)
    return _pool_call(p3, gp3, dinv, b3.reshape(1, D), batch3d, Wout,
                      bout.reshape(1, out_dim), out_dim)


# speedup vs baseline: 1.0176x; 1.0176x over previous
import numpy as _np

import jax
import jax.numpy as jnp
from jax import lax
from jax.experimental import pallas as pl
from jax.experimental.pallas import tpu as pltpu
from jax.experimental.pallas import tpu_sc as plsc

N = 10000
D = 128
E = 320000
B = 64

NC = 2
NS = 16
NW = NC * NS
CH = 128
NCH = 80
EPT = NCH * CH
PADE = NW * EPT
NP = 10240
RPT = NP // NS
ZR = 128

_mesh = plsc.VectorSubcoreMesh(core_axis_name="c", subcore_axis_name="s",
                               num_cores=NC, num_subcores=NS)

_PAD_SRC = _np.arange(PADE - E, dtype=_np.int32) % N
_PAD_DST = (N + _np.arange(PADE - E, dtype=_np.int32) % (NP - N)).astype(_np.int32)


def _zero_fill(ref, rows, width):
    per_row = width // 16

    def body(k, _):
        i = k // per_row
        j = k % per_row
        ref[i, pl.ds(j * 16, 16)] = jnp.zeros((16,), jnp.float32)
        return 0

    lax.fori_loop(0, rows * per_row, body, 0)


def _deg_body(dst2, degp, dst_all, ones_v, zbuf, acc, sem_d, sem_c):
    cid = lax.axis_index("c")
    sid = lax.axis_index("s")
    wid = cid * NS + sid

    pltpu.async_copy(dst2.at[pl.ds(wid * NCH, NCH), :], dst_all, sem_d)
    _zero_fill(zbuf, ZR, D)

    def ones_body(k, _):
        ones_v[k // 8, pl.ds((k % 8) * 16, 16)] = jnp.ones((16,), jnp.float32)
        return 0

    lax.fori_loop(0, CH * 8, ones_body, 0)
    for z in range(RPT // ZR):
        pltpu.sync_copy(zbuf, acc.at[pl.ds(sid * RPT + z * ZR, ZR)])
    plsc.subcore_barrier()
    pltpu.make_async_copy(dst2.at[pl.ds(wid * NCH, NCH), :], dst_all,
                          sem_d).wait()

    def grp(g, _):
        for k in range(8):
            pltpu.async_copy(ones_v, acc.at[dst_all.at[g * 8 + k]], sem_c,
                             add=True)
        for k in range(8):
            pltpu.make_async_copy(ones_v, acc.at[dst_all.at[g * 8 + k]],
                                  sem_c).wait()
        return 0

    lax.fori_loop(0, NCH // 8, grp, 0)

    plsc.subcore_barrier()
    for z in range(RPT // ZR):
        r0 = sid * RPT + z * ZR
        pltpu.sync_copy(acc.at[pl.ds(r0, ZR)], degp.at[cid, pl.ds(r0, ZR)])


_deg_call = pl.kernel(
    _deg_body,
    out_type=jax.ShapeDtypeStruct((NC, NP, D), jnp.float32),
    mesh=_mesh,
    scratch_types=[
        pltpu.VMEM((NCH, CH), jnp.int32),
        pltpu.VMEM((CH, D), jnp.float32),
        pltpu.VMEM((ZR, D), jnp.float32),
        pltpu.VMEM_SHARED((NP, D), jnp.float32),
        pltpu.SemaphoreType.DMA,
        pltpu.SemaphoreType.DMA,
    ],
)


def _agg_body(gp, src1, dst2, p, src_v0, src_v1, dst_all, rows0, rows1, acc,
              sem_d, sem_i0, sem_i1, sem_g0, sem_g1, sem_c0, sem_c1):
    cid = lax.axis_index("c")
    sid = lax.axis_index("s")
    wid = cid * NS + sid
    cb = wid * NCH
    eb = wid * EPT

    src_v = (src_v0, src_v1)
    sem_i = (sem_i0, sem_i1)
    rows = (rows0, rows1)
    sem_g = (sem_g0, sem_g1)
    sem_c = (sem_c0, sem_c1)

    def i_start(k, p_):
        pltpu.async_copy(src1.at[pl.ds(eb + k * CH, CH)], src_v[p_],
                         sem_i[p_])

    def i_wait(k, p_):
        pltpu.make_async_copy(src1.at[pl.ds(eb + k * CH, CH)], src_v[p_],
                              sem_i[p_]).wait()

    def g_start(k, p_):
        pltpu.async_copy(gp.at[src_v[p_]], rows[p_], sem_g[p_])

    def g_wait(k, p_):
        pltpu.make_async_copy(gp.at[src_v[p_]], rows[p_], sem_g[p_]).wait()

    def c_start(k, p_):
        pltpu.async_copy(rows[p_], acc.at[dst_all.at[k]], sem_c[p_], add=True)

    def c_wait(k, p_):
        pltpu.make_async_copy(rows[p_], acc.at[dst_all.at[k]],
                              sem_c[p_]).wait()

    pltpu.async_copy(dst2.at[pl.ds(cb, NCH), :], dst_all, sem_d)
    i_start(0, 0)
    _zero_fill(rows0, CH, D)
    for z in range(RPT // ZR):
        pltpu.sync_copy(rows0, acc.at[pl.ds(sid * RPT + z * ZR, ZR)])
    plsc.subcore_barrier()
    pltpu.make_async_copy(dst2.at[pl.ds(cb, NCH), :], dst_all, sem_d).wait()

    def c_sync(k, p_):
        pltpu.sync_copy(rows[p_], acc.at[dst_all.at[k]], add=True)

    i_wait(0, 0)
    g_start(0, 0)
    i_start(1, 1)
    g_wait(0, 0)
    i_wait(1, 1)
    g_start(1, 1)
    i_start(2, 0)
    c_sync(0, 0)

    def step(k, p_):
        g_wait(k, p_)
        i_wait(k + 1, 1 - p_)
        g_start(k + 1, 1 - p_)
        i_start(k + 2, p_)
        c_sync(k, p_)

    def body(g_, _):
        step(2 * g_ + 1, 1)
        step(2 * g_ + 2, 0)
        return 0

    lax.fori_loop(0, (NCH - 4) // 2, body, 0)
    step(NCH - 3, 1)
    k = NCH - 2
    g_wait(k, 0)
    i_wait(k + 1, 1)
    g_start(k + 1, 1)
    c_sync(k, 0)
    k = NCH - 1
    g_wait(k, 1)
    c_sync(k, 1)

    plsc.subcore_barrier()
    for z in range(RPT // ZR):
        r0 = sid * RPT + z * ZR
        pltpu.sync_copy(acc.at[pl.ds(r0, ZR)], p.at[cid, pl.ds(r0, ZR)])


_agg_call = pl.kernel(
    _agg_body,
    out_type=jax.ShapeDtypeStruct((NC, NP, D), jnp.float32),
    mesh=_mesh,
    scratch_types=[
        pltpu.VMEM((CH,), jnp.int32),
        pltpu.VMEM((CH,), jnp.int32),
        pltpu.VMEM((NCH, CH), jnp.int32),
        pltpu.VMEM((CH, D), jnp.float32),
        pltpu.VMEM((CH, D), jnp.float32),
        pltpu.VMEM_SHARED((NP, D), jnp.float32),
        pltpu.SemaphoreType.DMA,
        pltpu.SemaphoreType.DMA,
        pltpu.SemaphoreType.DMA,
        pltpu.SemaphoreType.DMA,
        pltpu.SemaphoreType.DMA,
        pltpu.SemaphoreType.DMA,
        pltpu.SemaphoreType.DMA,
    ],
)

MB = 1000


def _k0_body(x_ref, w_ref, degp_ref, gp_ref, dinv_ref):
    d = degp_ref[...]
    deg = 1.0 + d[0, :, 0:1] + d[1, :, 0:1]
    dinv = lax.rsqrt(deg)
    g = jnp.dot(x_ref[...], w_ref[...], preferred_element_type=jnp.float32)
    gp_ref[...] = g * dinv
    dinv_ref[...] = dinv


def _k0_call(x, w, degp):
    return pl.pallas_call(
        _k0_body,
        grid=(N // MB,),
        in_specs=[
            pl.BlockSpec((MB, D), lambda i: (i, 0)),
            pl.BlockSpec((D, D), lambda i: (0, 0)),
            pl.BlockSpec((NC, MB, D), lambda i: (0, i, 0)),
        ],
        out_specs=[
            pl.BlockSpec((MB, D), lambda i: (i, 0)),
            pl.BlockSpec((MB, 1), lambda i: (i, 0)),
        ],
        out_shape=[
            jax.ShapeDtypeStruct((N, D), jnp.float32),
            jax.ShapeDtypeStruct((N, 1), jnp.float32),
        ],
    )(x, w, degp)


def _mid_body(p_ref, gp_ref, dinv_ref, b_ref, w_ref, out_ref):
    pr = p_ref[...]
    dinv = dinv_ref[...]
    h = dinv * (pr[0] + pr[1] + gp_ref[...]) + b_ref[...]
    h = jnp.maximum(h, 0.0)
    out_ref[...] = dinv * jnp.dot(h, w_ref[...],
                                  preferred_element_type=jnp.float32)


def _mid_call(p, gp, dinv, b2d, w):
    return pl.pallas_call(
        _mid_body,
        grid=(N // MB,),
        in_specs=[
            pl.BlockSpec((NC, MB, D), lambda i: (0, i, 0)),
            pl.BlockSpec((MB, D), lambda i: (i, 0)),
            pl.BlockSpec((MB, 1), lambda i: (i, 0)),
            pl.BlockSpec((1, D), lambda i: (0, 0)),
            pl.BlockSpec((D, D), lambda i: (0, 0)),
        ],
        out_specs=pl.BlockSpec((MB, D), lambda i: (i, 0)),
        out_shape=jax.ShapeDtypeStruct((N, D), jnp.float32),
    )(p, gp, dinv, b2d, w)


PB = 400
PG = N // PB


def _pool_body(p_ref, gp_ref, dinv_ref, b_ref, batch_ref, wout_ref, bout_ref,
               out_ref, gmax_s, gsum_s, cnt_s):
    i = pl.program_id(0)

    @pl.when(i == 0)
    def _init():
        gmax_s[...] = jnp.full((B, D), -jnp.inf, jnp.float32)
        gsum_s[...] = jnp.zeros((B, D), jnp.float32)
        cnt_s[...] = jnp.zeros((B, 1), jnp.float32)

    pr = p_ref[...]
    dinv = dinv_ref[...]
    h = dinv * (pr[0] + pr[1] + gp_ref[...]) + b_ref[...]
    h = jnp.maximum(h, 0.0)

    bbc = batch_ref[0]
    gids = lax.broadcasted_iota(jnp.int32, (PB, B), 1)
    onehot = (gids == bbc).astype(jnp.float32)
    gsum_s[...] += lax.dot_general(
        onehot, h, (((0,), (0,)), ((), ())),
        preferred_element_type=jnp.float32)
    cnt_s[...] += jnp.sum(onehot, axis=0)[:, None]

    lo = bbc[0, 0]
    hi = bbc[PB - 1, 0]

    def seg(g, _):
        mask = bbc == g
        m = jnp.max(jnp.where(mask, h, -jnp.inf), axis=0, keepdims=True)
        cur = gmax_s[pl.ds(g, 1), :]
        gmax_s[pl.ds(g, 1), :] = jnp.maximum(cur, m)
        return 0

    lax.fori_loop(lo, hi + 1, seg, 0)

    @pl.when(i == PG - 1)
    def _final():
        gmean = gsum_s[...] / jnp.maximum(cnt_s[...], 1.0)
        pooled = jnp.concatenate([gmax_s[...], gmean], axis=1)
        out_ref[...] = jnp.dot(pooled, wout_ref[...],
                               preferred_element_type=jnp.float32) + bout_ref[...]


def _pool_call(p, gp, dinv, b2d, batch3d, wout, bout2d, out_dim):
    return pl.pallas_call(
        _pool_body,
        grid=(PG,),
        in_specs=[
            pl.BlockSpec((NC, PB, D), lambda i: (0, i, 0)),
            pl.BlockSpec((PB, D), lambda i: (i, 0)),
            pl.BlockSpec((PB, 1), lambda i: (i, 0)),
            pl.BlockSpec((1, D), lambda i: (0, 0)),
            pl.BlockSpec((1, PB, 1), lambda i: (i, 0, 0)),
            pl.BlockSpec((2 * D, out_dim), lambda i: (0, 0)),
            pl.BlockSpec((1, out_dim), lambda i: (0, 0)),
        ],
        out_specs=pl.BlockSpec((B, out_dim), lambda i: (0, 0)),
        out_shape=jax.ShapeDtypeStruct((B, out_dim), jnp.float32),
        scratch_shapes=[
            pltpu.VMEM((B, D), jnp.float32),
            pltpu.VMEM((B, D), jnp.float32),
            pltpu.VMEM((B, 1), jnp.float32),
        ],
    )(p, gp, dinv, b2d, batch3d, wout, bout2d)


def kernel(x, edge_index, batch_index, W0, b0, W1, b1, W2, b2, W3, b3,
           Wout, bout):
    out_dim = Wout.shape[1]
    batch3d = batch_index.reshape(PG, PB, 1)

    pad = PADE - E
    pad_dst = N + (jnp.arange(pad, dtype=jnp.int32) % (NP - N))
    dst2 = jnp.concatenate(
        [edge_index[1], pad_dst]).reshape(PADE // CH, CH)
    pad_src = jnp.arange(pad, dtype=jnp.int32) % N
    src1 = jnp.concatenate([edge_index[0], pad_src])
    degp = _deg_call(dst2)
    gp0, dinv = _k0_call(x, W0, degp)
    p0 = _agg_call(gp0, src1, dst2)
    gp1 = _mid_call(p0, gp0, dinv, b0.reshape(1, D), W1)
    p1 = _agg_call(gp1, src1, dst2)
    gp2 = _mid_call(p1, gp1, dinv, b1.reshape(1, D), W2)
    p2 = _agg_call(gp2, src1, dst2)
    gp3 = _mid_call(p2, gp2, dinv, b2.reshape(1, D), W3)
    p3 = _agg_call(gp3, src1, dst2)
    return _pool_call(p3, gp3, dinv, b3.reshape(1, D), batch3d, Wout,
                      bout.reshape(1, out_dim), out_dim)
